# baseline (speedup 1.0000x reference)
import functools

import jax
import jax.numpy as jnp
from jax import lax
from jax.experimental import pallas as pl
from jax.experimental.pallas import tpu as pltpu
from jax.experimental.pallas import tpu_sc as plsc

MASK_TOKEN = 103
LABEL_IGNORE = -100
RAND_THRESH_BITS = 0x3E19999A

ROWS = 4096
COLS = 2048
E = ROWS * COLS
NC, NS, LANES = 2, 16, 16
NW = NC * NS
PER_W = E // NW
CHUNK = 16384
NCH = PER_W // CHUNK
VECS = CHUNK // LANES


def _sc_body(ids_hbm, rand_hbm, m_hbm, l_hbm,
             buf_ids0, buf_rnd0, buf_ids1, buf_rnd1,
             si0, sr0, si1, sr1, som0, sol0, som1, sol1):
    wid = lax.axis_index("s") * NC + lax.axis_index("c")
    base = wid * PER_W

    bufs = ((buf_ids0, buf_rnd0, si0, sr0, som0, sol0),
            (buf_ids1, buf_rnd1, si1, sr1, som1, sol1))

    def start_in(c):
        ids_b, rnd_b, si, sr, _, _ = bufs[c % 2]
        off = base + c * CHUNK
        h_i = pltpu.async_copy(ids_hbm.at[pl.ds(off, CHUNK)], ids_b, si)
        h_r = pltpu.async_copy(rand_hbm.at[pl.ds(off, CHUNK)], rnd_b, sr)
        return h_i, h_r

    out_handles = [None, None]
    in_handles = [None, None]

    in_handles[0] = start_in(0)
    for c in range(NCH):
        b = c % 2
        ids_b, rnd_b, _, _, som, sol = bufs[b]
        if c + 1 < NCH:
            nb = (c + 1) % 2
            if out_handles[nb] is not None:
                out_handles[nb][0].wait()
                out_handles[nb][1].wait()
                out_handles[nb] = None
            in_handles[nb] = start_in(c + 1)
        in_handles[b][0].wait()
        in_handles[b][1].wait()

        @pl.loop(0, VECS, unroll=4)
        def _(i):
            sl = pl.ds(i * LANES, LANES)
            ids = ids_b[sl]
            rnd = rnd_b[sl]
            mask = rnd < RAND_THRESH_BITS
            masked = jnp.where(mask, jnp.int32(MASK_TOKEN), ids)
            labels = jnp.where(mask | (ids == MASK_TOKEN), ids,
                               jnp.int32(LABEL_IGNORE))
            rnd_b[sl] = masked
            ids_b[sl] = labels

        off = base + c * CHUNK
        h_m = pltpu.async_copy(rnd_b, m_hbm.at[pl.ds(off, CHUNK)], som)
        h_l = pltpu.async_copy(ids_b, l_hbm.at[pl.ds(off, CHUNK)], sol)
        out_handles[b] = (h_m, h_l)

    for b in range(2):
        if out_handles[b] is not None:
            out_handles[b][0].wait()
            out_handles[b][1].wait()


@jax.jit
def _sc_call(ids_flat, rand_bits_flat):
    mesh = plsc.VectorSubcoreMesh(core_axis_name="c", subcore_axis_name="s",
                                  num_cores=NC, num_subcores=NS)
    out = jax.ShapeDtypeStruct((E,), jnp.int32)
    vmem_i32 = functools.partial(pltpu.VMEM, (CHUNK,), jnp.int32)
    run = pl.kernel(
        _sc_body,
        out_type=(out, out),
        mesh=mesh,
        scratch_types=[vmem_i32() for _ in range(4)]
        + [pltpu.SemaphoreType.DMA for _ in range(8)],
    )
    return run(ids_flat, rand_bits_flat)


def kernel(input_ids, rand_vals):
    ids_flat = input_ids.reshape(E)
    rand_bits = lax.bitcast_convert_type(rand_vals, jnp.int32).reshape(E)
    masked, labels = _sc_call(ids_flat, rand_bits)
    return masked.reshape(ROWS, COLS), labels.reshape(ROWS, COLS)


# speedup vs baseline: 1.2692x; 1.2692x over previous
import functools

import jax
import jax.numpy as jnp
from jax import lax
from jax.experimental import pallas as pl
from jax.experimental.pallas import tpu as pltpu
from jax.experimental.pallas import tpu_sc as plsc

MASK_TOKEN = 103
LABEL_IGNORE = -100
RAND_THRESH_BITS = 0x3E19999A

ROWS = 4096
COLS = 2048
E = ROWS * COLS
NC, NS, LANES = 2, 16, 16
NW = NC * NS
PER_W = E // NW
CHUNK = 16384
NCH = PER_W // CHUNK
VECS = CHUNK // LANES


def _sc_body(ids_hbm, rand_hbm, m_hbm, l_hbm,
             buf_ids0, buf_rnd0, buf_ids1, buf_rnd1,
             si0, sr0, si1, sr1, som0, sol0, som1, sol1):
    wid = lax.axis_index("s") * NC + lax.axis_index("c")
    base = wid * PER_W

    bufs = ((buf_ids0, buf_rnd0, si0, sr0, som0, sol0),
            (buf_ids1, buf_rnd1, si1, sr1, som1, sol1))

    def start_in(c):
        ids_b, rnd_b, si, sr, _, _ = bufs[c % 2]
        off = base + c * CHUNK
        h_i = pltpu.async_copy(ids_hbm.at[pl.ds(off, CHUNK)], ids_b, si)
        h_r = pltpu.async_copy(rand_hbm.at[pl.ds(off, CHUNK)], rnd_b, sr)
        return h_i, h_r

    out_handles = [None, None]
    in_handles = [None, None]

    in_handles[0] = start_in(0)
    for c in range(NCH):
        b = c % 2
        ids_b, rnd_b, _, _, som, sol = bufs[b]
        if c + 1 < NCH:
            nb = (c + 1) % 2
            if out_handles[nb] is not None:
                out_handles[nb][0].wait()
                out_handles[nb][1].wait()
                out_handles[nb] = None
            in_handles[nb] = start_in(c + 1)
        in_handles[b][0].wait()
        in_handles[b][1].wait()

        @plsc.parallel_loop(0, VECS, unroll=8)
        def _(i):
            sl = pl.ds(i * LANES, LANES)
            ids = ids_b[sl]
            rnd = rnd_b[sl]
            mask = rnd < RAND_THRESH_BITS
            masked = jnp.where(mask, jnp.int32(MASK_TOKEN), ids)
            labels = jnp.where(masked == MASK_TOKEN, ids,
                               jnp.int32(LABEL_IGNORE))
            rnd_b[sl] = masked
            ids_b[sl] = labels

        off = base + c * CHUNK
        h_m = pltpu.async_copy(rnd_b, m_hbm.at[pl.ds(off, CHUNK)], som)
        h_l = pltpu.async_copy(ids_b, l_hbm.at[pl.ds(off, CHUNK)], sol)
        out_handles[b] = (h_m, h_l)

    for b in range(2):
        if out_handles[b] is not None:
            out_handles[b][0].wait()
            out_handles[b][1].wait()


@jax.jit
def _sc_call(ids_flat, rand_bits_flat):
    mesh = plsc.VectorSubcoreMesh(core_axis_name="c", subcore_axis_name="s",
                                  num_cores=NC, num_subcores=NS)
    out = jax.ShapeDtypeStruct((E,), jnp.int32)
    vmem_i32 = functools.partial(pltpu.VMEM, (CHUNK,), jnp.int32)
    run = pl.kernel(
        _sc_body,
        out_type=(out, out),
        mesh=mesh,
        scratch_types=[vmem_i32() for _ in range(4)]
        + [pltpu.SemaphoreType.DMA for _ in range(8)],
    )
    return run(ids_flat, rand_bits_flat)


def kernel(input_ids, rand_vals):
    ids_flat = input_ids.reshape(E)
    rand_bits = lax.bitcast_convert_type(rand_vals, jnp.int32).reshape(E)
    masked, labels = _sc_call(ids_flat, rand_bits)
    return masked.reshape(ROWS, COLS), labels.reshape(ROWS, COLS)


# speedup vs baseline: 3.0721x; 2.4205x over previous
import jax
import jax.numpy as jnp
from jax import lax
from jax.experimental import pallas as pl
from jax.experimental.pallas import tpu as pltpu
from jax.experimental.pallas import tpu_sc as plsc

MASK_TOKEN = 103
LABEL_IGNORE = -100
RAND_THRESH_BITS = 0x3E19999A

ROWS = 4096
COLS = 2048
NC, NS, LANES = 2, 16, 16
NW = NC * NS
ROWS_W = ROWS // NW
CR = 8
NCH = ROWS_W // CR
VECS = CR * COLS // LANES


def _sc_body(ids_hbm, rand_hbm, m_hbm, l_hbm,
             buf_ids0, buf_rnd0, buf_ids1, buf_rnd1,
             si0, sr0, si1, sr1, som0, sol0, som1, sol1):
    wid = lax.axis_index("s") * NC + lax.axis_index("c")
    base = wid * ROWS_W

    bufs = ((buf_ids0, buf_rnd0, si0, sr0, som0, sol0),
            (buf_ids1, buf_rnd1, si1, sr1, som1, sol1))

    def start_in(c):
        ids_b, rnd_b, si, sr, _, _ = bufs[c % 2]
        r0 = base + c * CR
        h_i = pltpu.async_copy(ids_hbm.at[pl.ds(r0, CR)], ids_b, si)
        h_r = pltpu.async_copy(rand_hbm.at[pl.ds(r0, CR)], rnd_b, sr)
        return h_i, h_r

    out_handles = [None, None]
    in_handles = [None, None]

    in_handles[0] = start_in(0)
    for c in range(NCH):
        b = c % 2
        ids_b, rnd_b, _, _, som, sol = bufs[b]
        if c + 1 < NCH:
            nb = (c + 1) % 2
            if out_handles[nb] is not None:
                out_handles[nb][0].wait()
                out_handles[nb][1].wait()
                out_handles[nb] = None
            in_handles[nb] = start_in(c + 1)
        in_handles[b][0].wait()
        in_handles[b][1].wait()

        @plsc.parallel_loop(0, VECS, unroll=8)
        def _(i):
            r = i >> 7
            sl = pl.ds((i & 127) * LANES, LANES)
            ids = ids_b[r, sl]
            rnd = rnd_b[r, sl]
            mask = rnd < RAND_THRESH_BITS
            masked = jnp.where(mask, jnp.int32(MASK_TOKEN), ids)
            labels = jnp.where(masked == MASK_TOKEN, ids,
                               jnp.int32(LABEL_IGNORE))
            rnd_b[r, sl] = masked
            ids_b[r, sl] = labels

        r0 = base + c * CR
        h_m = pltpu.async_copy(rnd_b, m_hbm.at[pl.ds(r0, CR)], som)
        h_l = pltpu.async_copy(ids_b, l_hbm.at[pl.ds(r0, CR)], sol)
        out_handles[b] = (h_m, h_l)

    for b in range(2):
        if out_handles[b] is not None:
            out_handles[b][0].wait()
            out_handles[b][1].wait()


@jax.jit
def _sc_call(ids, rand_bits):
    mesh = plsc.VectorSubcoreMesh(core_axis_name="c", subcore_axis_name="s",
                                  num_cores=NC, num_subcores=NS)
    out = jax.ShapeDtypeStruct((ROWS, COLS), jnp.int32)
    run = pl.kernel(
        _sc_body,
        out_type=(out, out),
        mesh=mesh,
        scratch_types=[pltpu.VMEM((CR, COLS), jnp.int32) for _ in range(4)]
        + [pltpu.SemaphoreType.DMA for _ in range(8)],
    )
    return run(ids, rand_bits)


def kernel(input_ids, rand_vals):
    rand_bits = lax.bitcast_convert_type(rand_vals, jnp.int32)
    return _sc_call(input_ids, rand_bits)
